# baseline (speedup 1.0000x reference)
import functools

import jax
import jax.numpy as jnp
from jax import lax
from jax.experimental import pallas as pl
from jax.experimental.pallas import tpu as pltpu
from jax.experimental.pallas import tpu_sc as plsc

H = 128
N_MOLS = 1000
MOL = 50
NA1 = N_MOLS * MOL + 1
NB1 = 150001
MAXNB = 6

NAp = 51200
NBp = 151552

NW = 32
APW = NAp // NW
AC = 64
ANC = APW // AC
AIR = (AC * MAXNB) // 128
BPW = NBp // NW
BC = 128
BNC = BPW // BC

_SC_MESH = plsc.VectorSubcoreMesh(core_axis_name="c", subcore_axis_name="s",
                                  num_cores=2, num_subcores=16)


def _wid():
    return lax.axis_index("s") * 2 + lax.axis_index("c")


def _sc_agg_body(add, bond_hbm, a2b_hbm, atom_hbm, out_hbm,
                 idx_v, gbuf0, gbuf1, abuf, obuf, sem0, sem1, sem2):
    w = _wid()
    pltpu.sync_copy(a2b_hbm.at[w], idx_v)
    gbufs = (gbuf0, gbuf1)
    sems = (sem0, sem1)

    def start(c, b):
        return [pltpu.async_copy(bond_hbm.at[idx_v.at[c * AIR + j]],
                                 gbufs[b].at[pl.ds(j * 128, 128)], sems[b])
                for j in range(AIR)]

    def finish(c, b):
        for j in range(AIR):
            pltpu.make_async_copy(bond_hbm.at[idx_v.at[c * AIR + j]],
                                  gbufs[b].at[pl.ds(j * 128, 128)],
                                  sems[b]).wait()

    def compute(c, b):
        abase = w * APW + c * AC
        if add:
            pltpu.async_copy(atom_hbm.at[pl.ds(abase, AC)], abuf, sem2).wait()
        gbuf = gbufs[b]

        def atom_body(a, carry2):
            r0 = a * MAXNB
            for c8 in range(H // 16):
                sl = pl.ds(c8 * 16, 16)
                v0 = gbuf[r0 + 0, sl]
                v1 = gbuf[r0 + 1, sl]
                v2 = gbuf[r0 + 2, sl]
                v3 = gbuf[r0 + 3, sl]
                v4 = gbuf[r0 + 4, sl]
                v5 = gbuf[r0 + 5, sl]
                s = ((v0 + v4) + v2) + ((v1 + v5) + v3)
                m = jnp.maximum(jnp.maximum(jnp.maximum(v0, v1),
                                            jnp.maximum(v2, v3)),
                                jnp.maximum(v4, v5))
                o = s * m
                if add:
                    o = o + abuf[a, sl]
                obuf[a, sl] = o
            return carry2

        lax.fori_loop(0, AC, atom_body, 0)
        pltpu.sync_copy(obuf, out_hbm.at[pl.ds(abase, AC)])

    start(0, 0)

    def pair(i, carry):
        c0 = 2 * i
        c1 = c0 + 1

        @pl.when(c1 < ANC)
        def _():
            start(c1, 1)
        finish(c0, 0)
        compute(c0, 0)

        @pl.when(c0 + 2 < ANC)
        def _():
            start(c0 + 2, 0)

        @pl.when(c1 < ANC)
        def _():
            finish(c1, 1)
            compute(c1, 1)
        return carry

    lax.fori_loop(0, (ANC + 1) // 2, pair, 0)


def _make_sc_agg(add):
    body = functools.partial(_sc_agg_body, add)
    return pl.kernel(
        body,
        out_type=jax.ShapeDtypeStruct((NAp, H), jnp.float32),
        mesh=_SC_MESH,
        scratch_types=[
            pltpu.VMEM((ANC * AIR, 128), jnp.int32),
            pltpu.VMEM((AIR * 128, H), jnp.float32),
            pltpu.VMEM((AIR * 128, H), jnp.float32),
            pltpu.VMEM((AC, H), jnp.float32),
            pltpu.VMEM((AC, H), jnp.float32),
            pltpu.SemaphoreType.DMA,
            pltpu.SemaphoreType.DMA,
            pltpu.SemaphoreType.DMA,
        ],
    )


def _sc_bond_gather_body(atom_hbm, bond_hbm, b2a_hbm, b2revb_hbm,
                         o1_hbm, o2_hbm, ia, ir,
                         g10, g20, g11, g21, sem0, sem1, osem0, osem1):
    w = _wid()
    pltpu.sync_copy(b2a_hbm.at[w], ia)
    pltpu.sync_copy(b2revb_hbm.at[w], ir)
    g1s = (g10, g11)
    g2s = (g20, g21)
    sems = (sem0, sem1)
    osems = (osem0, osem1)

    def start(c, b):
        pltpu.async_copy(atom_hbm.at[ia.at[c]], g1s[b], sems[b])
        pltpu.async_copy(bond_hbm.at[ir.at[c]], g2s[b], sems[b])

    def finish_out(c, b):
        bbase = w * BPW + c * BC
        pltpu.make_async_copy(g1s[b], o1_hbm.at[pl.ds(bbase, BC)], osems[b]).wait()
        pltpu.make_async_copy(g2s[b], o2_hbm.at[pl.ds(bbase, BC)], osems[b]).wait()

    def emit(c, b):
        bbase = w * BPW + c * BC
        pltpu.make_async_copy(atom_hbm.at[ia.at[c]], g1s[b], sems[b]).wait()
        pltpu.make_async_copy(bond_hbm.at[ir.at[c]], g2s[b], sems[b]).wait()
        pltpu.async_copy(g1s[b], o1_hbm.at[pl.ds(bbase, BC)], osems[b])
        pltpu.async_copy(g2s[b], o2_hbm.at[pl.ds(bbase, BC)], osems[b])

    start(0, 0)
    start(1, 1)

    def pair(i, carry):
        c0 = 2 * i
        c1 = c0 + 1
        emit(c0, 0)

        @pl.when(c0 + 2 < BNC)
        def _():
            finish_out(c0, 0)
            start(c0 + 2, 0)

        @pl.when(c1 < BNC)
        def _():
            emit(c1, 1)

        @pl.when(c1 + 2 < BNC)
        def _():
            finish_out(c1, 1)
            start(c1 + 2, 1)
        return carry

    lax.fori_loop(0, (BNC + 1) // 2, pair, 0)
    finish_out(BNC - 2, (BNC - 2) % 2)
    finish_out(BNC - 1, (BNC - 1) % 2)


_sc_bond_gather = pl.kernel(
    _sc_bond_gather_body,
    out_type=(jax.ShapeDtypeStruct((NBp, H), jnp.float32),
              jax.ShapeDtypeStruct((NBp, H), jnp.float32)),
    mesh=_SC_MESH,
    scratch_types=[
        pltpu.VMEM((BNC, 128), jnp.int32),
        pltpu.VMEM((BNC, 128), jnp.int32),
        pltpu.VMEM((BC, H), jnp.float32),
        pltpu.VMEM((BC, H), jnp.float32),
        pltpu.VMEM((BC, H), jnp.float32),
        pltpu.VMEM((BC, H), jnp.float32),
        pltpu.SemaphoreType.DMA,
        pltpu.SemaphoreType.DMA,
        pltpu.SemaphoreType.DMA,
        pltpu.SemaphoreType.DMA,
    ],
)


def _dotT(x, w):
    return lax.dot_general(x, w, (((1,), (1,)), ((), ())),
                           preferred_element_type=jnp.float32)


def _proj_relu(x, w, np_, bm):
    n, k = x.shape
    last = n // bm - 1

    def body(x_ref, w_ref, o_ref):
        o_ref[...] = jax.nn.relu(_dotT(x_ref[...], w_ref[...]))

    return pl.pallas_call(
        body,
        grid=(np_ // bm,),
        in_specs=[pl.BlockSpec((bm, k), lambda i: (jnp.minimum(i, last), 0)),
                  pl.BlockSpec((H, k), lambda i: (0, 0))],
        out_specs=pl.BlockSpec((bm, H), lambda i: (i, 0)),
        out_shape=jax.ShapeDtypeStruct((np_, H), jnp.float32),
    )(x, w)


def _bond_update(g1, g2, ib, w, bm=1024):
    def body(g1_ref, g2_ref, ib_ref, w_ref, o_ref):
        mb = g1_ref[...] - g2_ref[...]
        o_ref[...] = jax.nn.relu(ib_ref[...] + _dotT(mb, w_ref[...]))

    return pl.pallas_call(
        body,
        grid=(NBp // bm,),
        in_specs=[pl.BlockSpec((bm, H), lambda i: (i, 0)),
                  pl.BlockSpec((bm, H), lambda i: (i, 0)),
                  pl.BlockSpec((bm, H), lambda i: (i, 0)),
                  pl.BlockSpec((H, H), lambda i: (0, 0))],
        out_specs=pl.BlockSpec((bm, H), lambda i: (i, 0)),
        out_shape=jax.ShapeDtypeStruct((NBp, H), jnp.float32),
    )(g1, g2, ib, w)


def _lr_matmul(agg, ma, ia, lr_w, bm=1024):
    def body(a_ref, m_ref, i_ref, w_ref, o_ref):
        cat = jnp.concatenate([a_ref[...], m_ref[...], i_ref[...]], axis=1)
        o_ref[...] = _dotT(cat, w_ref[...])

    return pl.pallas_call(
        body,
        grid=(NAp // bm,),
        in_specs=[pl.BlockSpec((bm, H), lambda i: (i, 0)),
                  pl.BlockSpec((bm, H), lambda i: (i, 0)),
                  pl.BlockSpec((bm, H), lambda i: (i, 0)),
                  pl.BlockSpec((H, 3 * H), lambda i: (0, 0))],
        out_specs=pl.BlockSpec((bm, H), lambda i: (i, 0)),
        out_shape=jax.ShapeDtypeStruct((NAp, H), jnp.float32),
    )(agg, ma, ia, lr_w)


def _gru_readout(node_t, gbias, wih_f, whh_f, bih_f, bhh_f,
                 wih_b, whh_b, bih_b, bhh_b, wo, wob, mb=500):
    def body(nd_ref, gb_ref, wif_ref, whf_ref, bif_ref, bhf_ref,
             wib_ref, whb_ref, bib_ref, bhb_ref, wo_ref, wob_ref,
             o_ref, fwd_ref):
        gb = gb_ref[...]

        def xt_at(t):
            x = jnp.reshape(nd_ref[:, pl.ds(t, 1), :], (mb, H))
            return jax.nn.relu(x + gb)

        def gru_step(xt, h, wi, wh, bi, bh):
            gi = _dotT(xt, wi[...]) + bi[...]
            gh = _dotT(h, wh[...]) + bh[...]
            r = jax.nn.sigmoid(gi[:, 0:H] + gh[:, 0:H])
            z = jax.nn.sigmoid(gi[:, H:2 * H] + gh[:, H:2 * H])
            n = jnp.tanh(gi[:, 2 * H:3 * H] + r * gh[:, 2 * H:3 * H])
            return (1.0 - z) * n + z * h

        h0 = jnp.max(nd_ref[...], axis=1)

        def fstep(t, h):
            h2 = gru_step(xt_at(t), h, wif_ref, whf_ref, bif_ref, bhf_ref)
            fwd_ref[:, pl.ds(t, 1), :] = jnp.reshape(h2, (mb, 1, H))
            return h2

        lax.fori_loop(0, MOL, fstep, h0)

        wo = wo_ref[...]
        wob = wob_ref[...]

        def bstep(i, carry):
            h, acc = carry
            t = MOL - 1 - i
            h2 = gru_step(xt_at(t), h, wib_ref, whb_ref, bib_ref, bhb_ref)
            fwd_t = jnp.reshape(fwd_ref[:, pl.ds(t, 1), :], (mb, H))
            cat = jnp.concatenate([fwd_t, h2], axis=1)
            ah = jax.nn.relu(_dotT(cat, wo) + wob)
            return h2, acc + ah

        _, acc = lax.fori_loop(0, MOL, bstep,
                               (h0, jnp.zeros((mb, H), jnp.float32)))
        o_ref[...] = jnp.reshape(acc * (1.0 / MOL), (1, mb, H))

    return pl.pallas_call(
        body,
        grid=(N_MOLS // mb,),
        in_specs=[pl.BlockSpec((mb, MOL, H), lambda i: (i, 0, 0)),
                  pl.BlockSpec((1, H), lambda i: (0, 0)),
                  pl.BlockSpec((3 * H, H), lambda i: (0, 0)),
                  pl.BlockSpec((3 * H, H), lambda i: (0, 0)),
                  pl.BlockSpec((1, 3 * H), lambda i: (0, 0)),
                  pl.BlockSpec((1, 3 * H), lambda i: (0, 0)),
                  pl.BlockSpec((3 * H, H), lambda i: (0, 0)),
                  pl.BlockSpec((3 * H, H), lambda i: (0, 0)),
                  pl.BlockSpec((1, 3 * H), lambda i: (0, 0)),
                  pl.BlockSpec((1, 3 * H), lambda i: (0, 0)),
                  pl.BlockSpec((H, 2 * H), lambda i: (0, 0)),
                  pl.BlockSpec((1, H), lambda i: (0, 0))],
        out_specs=pl.BlockSpec((1, mb, H), lambda i: (i, 0, 0)),
        out_shape=jax.ShapeDtypeStruct((N_MOLS // mb, mb, H), jnp.float32),
        scratch_shapes=[pltpu.VMEM((mb, MOL, H), jnp.float32)],
    )(node_t, gbias, wih_f, whh_f, bih_f, bhh_f,
      wih_b, whh_b, bih_b, bhh_b, wo, wob)


def kernel(f_atoms, f_bonds, a2b, b2a, b2revb, a_scope, params):
    p = params

    a2b_p = jnp.pad(a2b, ((0, NAp - NA1), (0, 0)))
    a2b2d = a2b_p.reshape(NW, APW * MAXNB // 128, 128)
    b2a2d = jnp.pad(b2a, (0, NBp - NB1)).reshape(NW, BNC, 128)
    b2revb2d = jnp.pad(b2revb, (0, NBp - NB1)).reshape(NW, BNC, 128)

    fa = jnp.pad(f_atoms, ((0, -NA1 % 1024), (0, 0)))
    fb = jnp.pad(f_bonds, ((0, -NB1 % 1024), (0, 0)))
    input_atom = _proj_relu(fa, p['W_i_atom'], NAp, bm=1024)
    input_bond = _proj_relu(fb, p['W_i_bond'], NBp, bm=1024)

    sc_agg_add = _make_sc_agg(True)
    sc_agg = _make_sc_agg(False)

    message_atom = input_atom
    message_bond = input_bond
    for d in range(2):
        message_atom = sc_agg_add(message_bond, a2b2d, message_atom)
        g1, g2 = _sc_bond_gather(message_atom, message_bond, b2a2d, b2revb2d)
        message_bond = _bond_update(g1, g2, input_bond, p['W_h_%d' % d])

    agg_f = sc_agg(message_bond, a2b2d, message_atom)
    node = _lr_matmul(agg_f, message_atom, input_atom, p['lr'])

    node3 = node[1:NA1].reshape(N_MOLS, MOL, H)
    mol_vecs = _gru_readout(
        node3,
        p['gru_bias'].reshape(1, H),
        p['gru_Wih_f'], p['gru_Whh_f'],
        p['gru_bih_f'].reshape(1, 3 * H), p['gru_bhh_f'].reshape(1, 3 * H),
        p['gru_Wih_b'], p['gru_Whh_b'],
        p['gru_bih_b'].reshape(1, 3 * H), p['gru_bhh_b'].reshape(1, 3 * H),
        p['W_o_w'], p['W_o_b'].reshape(1, H),
    )
    return mol_vecs.reshape(N_MOLS, H)


# speedup vs baseline: 1.2397x; 1.2397x over previous
import functools

import jax
import jax.numpy as jnp
from jax import lax
from jax.experimental import pallas as pl
from jax.experimental.pallas import tpu as pltpu
from jax.experimental.pallas import tpu_sc as plsc

H = 128
N_MOLS = 1000
MOL = 50
NA1 = N_MOLS * MOL + 1
NB1 = 150001
MAXNB = 6

NAp = 51200
NBp = 151552

NW = 32
APW = NAp // NW
AC = 64
ANC = APW // AC
AIR = (AC * MAXNB) // 128
BPW = NBp // NW
BC = 128
BNC = BPW // BC

_SC_MESH = plsc.VectorSubcoreMesh(core_axis_name="c", subcore_axis_name="s",
                                  num_cores=2, num_subcores=16)


def _wid():
    return lax.axis_index("s") * 2 + lax.axis_index("c")


def _sc_agg_body(add, bond_hbm, a2b_hbm, atom_hbm, out_hbm,
                 idx_v, gbuf0, gbuf1, abuf, obuf, sem0, sem1, sem2):
    w = _wid()
    pltpu.sync_copy(a2b_hbm.at[w], idx_v)
    gbufs = (gbuf0, gbuf1)
    sems = (sem0, sem1)

    def start(c, b):
        return [pltpu.async_copy(bond_hbm.at[idx_v.at[c * AIR + j]],
                                 gbufs[b].at[pl.ds(j * 128, 128)], sems[b])
                for j in range(AIR)]

    def finish(c, b):
        for j in range(AIR):
            pltpu.make_async_copy(bond_hbm.at[idx_v.at[c * AIR + j]],
                                  gbufs[b].at[pl.ds(j * 128, 128)],
                                  sems[b]).wait()

    def compute(c, b):
        abase = w * APW + c * AC
        if add:
            pltpu.async_copy(atom_hbm.at[pl.ds(abase, AC)], abuf, sem2).wait()
        gbuf = gbufs[b]

        @plsc.parallel_loop(0, AC, step=1, unroll=4)
        def atom_body(a):
            r0 = a * MAXNB
            for c8 in range(H // 16):
                sl = pl.ds(c8 * 16, 16)
                v0 = gbuf[r0 + 0, sl]
                v1 = gbuf[r0 + 1, sl]
                v2 = gbuf[r0 + 2, sl]
                v3 = gbuf[r0 + 3, sl]
                v4 = gbuf[r0 + 4, sl]
                v5 = gbuf[r0 + 5, sl]
                s = ((v0 + v4) + v2) + ((v1 + v5) + v3)
                m = jnp.maximum(jnp.maximum(jnp.maximum(v0, v1),
                                            jnp.maximum(v2, v3)),
                                jnp.maximum(v4, v5))
                o = s * m
                if add:
                    o = o + abuf[a, sl]
                obuf[a, sl] = o
        pltpu.sync_copy(obuf, out_hbm.at[pl.ds(abase, AC)])

    start(0, 0)

    def pair(i, carry):
        c0 = 2 * i
        c1 = c0 + 1

        @pl.when(c1 < ANC)
        def _():
            start(c1, 1)
        finish(c0, 0)
        compute(c0, 0)

        @pl.when(c0 + 2 < ANC)
        def _():
            start(c0 + 2, 0)

        @pl.when(c1 < ANC)
        def _():
            finish(c1, 1)
            compute(c1, 1)
        return carry

    lax.fori_loop(0, (ANC + 1) // 2, pair, 0)


def _make_sc_agg(add):
    body = functools.partial(_sc_agg_body, add)
    return pl.kernel(
        body,
        out_type=jax.ShapeDtypeStruct((NAp, H), jnp.float32),
        mesh=_SC_MESH,
        scratch_types=[
            pltpu.VMEM((ANC * AIR, 128), jnp.int32),
            pltpu.VMEM((AIR * 128, H), jnp.float32),
            pltpu.VMEM((AIR * 128, H), jnp.float32),
            pltpu.VMEM((AC, H), jnp.float32),
            pltpu.VMEM((AC, H), jnp.float32),
            pltpu.SemaphoreType.DMA,
            pltpu.SemaphoreType.DMA,
            pltpu.SemaphoreType.DMA,
        ],
    )


def _sc_bond_gather_body(atom_hbm, bond_hbm, b2a_hbm, b2revb_hbm,
                         o1_hbm, o2_hbm, ia, ir,
                         g10, g20, g11, g21, sem0, sem1, osem0, osem1):
    w = _wid()
    pltpu.sync_copy(b2a_hbm.at[w], ia)
    pltpu.sync_copy(b2revb_hbm.at[w], ir)
    g1s = (g10, g11)
    g2s = (g20, g21)
    sems = (sem0, sem1)
    osems = (osem0, osem1)

    def start(c, b):
        pltpu.async_copy(atom_hbm.at[ia.at[c]], g1s[b], sems[b])
        pltpu.async_copy(bond_hbm.at[ir.at[c]], g2s[b], sems[b])

    def finish_out(c, b):
        bbase = w * BPW + c * BC
        pltpu.make_async_copy(g1s[b], o1_hbm.at[pl.ds(bbase, BC)], osems[b]).wait()
        pltpu.make_async_copy(g2s[b], o2_hbm.at[pl.ds(bbase, BC)], osems[b]).wait()

    def emit(c, b):
        bbase = w * BPW + c * BC
        pltpu.make_async_copy(atom_hbm.at[ia.at[c]], g1s[b], sems[b]).wait()
        pltpu.make_async_copy(bond_hbm.at[ir.at[c]], g2s[b], sems[b]).wait()
        pltpu.async_copy(g1s[b], o1_hbm.at[pl.ds(bbase, BC)], osems[b])
        pltpu.async_copy(g2s[b], o2_hbm.at[pl.ds(bbase, BC)], osems[b])

    start(0, 0)
    start(1, 1)

    def pair(i, carry):
        c0 = 2 * i
        c1 = c0 + 1
        emit(c0, 0)

        @pl.when(c0 + 2 < BNC)
        def _():
            finish_out(c0, 0)
            start(c0 + 2, 0)

        @pl.when(c1 < BNC)
        def _():
            emit(c1, 1)

        @pl.when(c1 + 2 < BNC)
        def _():
            finish_out(c1, 1)
            start(c1 + 2, 1)
        return carry

    lax.fori_loop(0, (BNC + 1) // 2, pair, 0)
    finish_out(BNC - 2, (BNC - 2) % 2)
    finish_out(BNC - 1, (BNC - 1) % 2)


_sc_bond_gather = pl.kernel(
    _sc_bond_gather_body,
    out_type=(jax.ShapeDtypeStruct((NBp, H), jnp.float32),
              jax.ShapeDtypeStruct((NBp, H), jnp.float32)),
    mesh=_SC_MESH,
    scratch_types=[
        pltpu.VMEM((BNC, 128), jnp.int32),
        pltpu.VMEM((BNC, 128), jnp.int32),
        pltpu.VMEM((BC, H), jnp.float32),
        pltpu.VMEM((BC, H), jnp.float32),
        pltpu.VMEM((BC, H), jnp.float32),
        pltpu.VMEM((BC, H), jnp.float32),
        pltpu.SemaphoreType.DMA,
        pltpu.SemaphoreType.DMA,
        pltpu.SemaphoreType.DMA,
        pltpu.SemaphoreType.DMA,
    ],
)


def _dotT(x, w):
    return lax.dot_general(x, w, (((1,), (1,)), ((), ())),
                           preferred_element_type=jnp.float32)


def _proj_relu(x, w, np_, bm):
    n, k = x.shape
    nmain = n // bm
    tail = jnp.zeros((np_ - nmain * bm, k), x.dtype).at[:n - nmain * bm].set(
        x[nmain * bm:])

    def body(x_ref, t_ref, w_ref, o_ref):
        i = pl.program_id(0)

        @pl.when(i < nmain)
        def _():
            o_ref[...] = jax.nn.relu(_dotT(x_ref[...], w_ref[...]))

        @pl.when(i >= nmain)
        def _():
            o_ref[...] = jax.nn.relu(_dotT(t_ref[...], w_ref[...]))

    return pl.pallas_call(
        body,
        grid=(np_ // bm,),
        in_specs=[
            pl.BlockSpec((bm, k), lambda i: (jnp.minimum(i, nmain - 1), 0)),
            pl.BlockSpec((bm, k), lambda i: (jnp.maximum(i - nmain, 0), 0)),
            pl.BlockSpec((H, k), lambda i: (0, 0)),
        ],
        out_specs=pl.BlockSpec((bm, H), lambda i: (i, 0)),
        out_shape=jax.ShapeDtypeStruct((np_, H), jnp.float32),
    )(x, tail, w)


def _bond_update(g1, g2, ib, w, bm=1024):
    def body(g1_ref, g2_ref, ib_ref, w_ref, o_ref):
        mb = g1_ref[...] - g2_ref[...]
        o_ref[...] = jax.nn.relu(ib_ref[...] + _dotT(mb, w_ref[...]))

    return pl.pallas_call(
        body,
        grid=(NBp // bm,),
        in_specs=[pl.BlockSpec((bm, H), lambda i: (i, 0)),
                  pl.BlockSpec((bm, H), lambda i: (i, 0)),
                  pl.BlockSpec((bm, H), lambda i: (i, 0)),
                  pl.BlockSpec((H, H), lambda i: (0, 0))],
        out_specs=pl.BlockSpec((bm, H), lambda i: (i, 0)),
        out_shape=jax.ShapeDtypeStruct((NBp, H), jnp.float32),
    )(g1, g2, ib, w)


def _lr_matmul(agg, ma, ia, lr_w, bm=1024):
    def body(a_ref, m_ref, i_ref, w_ref, o_ref):
        cat = jnp.concatenate([a_ref[...], m_ref[...], i_ref[...]], axis=1)
        o_ref[...] = _dotT(cat, w_ref[...])

    return pl.pallas_call(
        body,
        grid=(NAp // bm,),
        in_specs=[pl.BlockSpec((bm, H), lambda i: (i, 0)),
                  pl.BlockSpec((bm, H), lambda i: (i, 0)),
                  pl.BlockSpec((bm, H), lambda i: (i, 0)),
                  pl.BlockSpec((H, 3 * H), lambda i: (0, 0))],
        out_specs=pl.BlockSpec((bm, H), lambda i: (i, 0)),
        out_shape=jax.ShapeDtypeStruct((NAp, H), jnp.float32),
    )(agg, ma, ia, lr_w)


def _gru_readout(node_t, gbias, wih_f, whh_f, bih_f, bhh_f,
                 wih_b, whh_b, bih_b, bhh_b, wo, wob, mb=500):
    def body(nd_ref, gb_ref, wif_ref, whf_ref, bif_ref, bhf_ref,
             wib_ref, whb_ref, bib_ref, bhb_ref, wo_ref, wob_ref,
             o_ref, fwd_ref):
        gb = gb_ref[...]

        def xt_at(t):
            x = jnp.reshape(nd_ref[:, pl.ds(t, 1), :], (mb, H))
            return jax.nn.relu(x + gb)

        def gru_step(xt, h, wi, wh, bi, bh):
            gi = _dotT(xt, wi[...]) + bi[...]
            gh = _dotT(h, wh[...]) + bh[...]
            r = jax.nn.sigmoid(gi[:, 0:H] + gh[:, 0:H])
            z = jax.nn.sigmoid(gi[:, H:2 * H] + gh[:, H:2 * H])
            n = jnp.tanh(gi[:, 2 * H:3 * H] + r * gh[:, 2 * H:3 * H])
            return (1.0 - z) * n + z * h

        h0 = jnp.max(nd_ref[...], axis=1)

        def fstep(t, h):
            h2 = gru_step(xt_at(t), h, wif_ref, whf_ref, bif_ref, bhf_ref)
            fwd_ref[:, pl.ds(t, 1), :] = jnp.reshape(h2, (mb, 1, H))
            return h2

        lax.fori_loop(0, MOL, fstep, h0)

        wo = wo_ref[...]
        wob = wob_ref[...]

        def bstep(i, carry):
            h, acc = carry
            t = MOL - 1 - i
            h2 = gru_step(xt_at(t), h, wib_ref, whb_ref, bib_ref, bhb_ref)
            fwd_t = jnp.reshape(fwd_ref[:, pl.ds(t, 1), :], (mb, H))
            cat = jnp.concatenate([fwd_t, h2], axis=1)
            ah = jax.nn.relu(_dotT(cat, wo) + wob)
            return h2, acc + ah

        _, acc = lax.fori_loop(0, MOL, bstep,
                               (h0, jnp.zeros((mb, H), jnp.float32)))
        o_ref[...] = jnp.reshape(acc * (1.0 / MOL), (1, mb, H))

    return pl.pallas_call(
        body,
        grid=(N_MOLS // mb,),
        in_specs=[pl.BlockSpec((mb, MOL, H), lambda i: (i, 0, 0)),
                  pl.BlockSpec((1, H), lambda i: (0, 0)),
                  pl.BlockSpec((3 * H, H), lambda i: (0, 0)),
                  pl.BlockSpec((3 * H, H), lambda i: (0, 0)),
                  pl.BlockSpec((1, 3 * H), lambda i: (0, 0)),
                  pl.BlockSpec((1, 3 * H), lambda i: (0, 0)),
                  pl.BlockSpec((3 * H, H), lambda i: (0, 0)),
                  pl.BlockSpec((3 * H, H), lambda i: (0, 0)),
                  pl.BlockSpec((1, 3 * H), lambda i: (0, 0)),
                  pl.BlockSpec((1, 3 * H), lambda i: (0, 0)),
                  pl.BlockSpec((H, 2 * H), lambda i: (0, 0)),
                  pl.BlockSpec((1, H), lambda i: (0, 0))],
        out_specs=pl.BlockSpec((1, mb, H), lambda i: (i, 0, 0)),
        out_shape=jax.ShapeDtypeStruct((N_MOLS // mb, mb, H), jnp.float32),
        scratch_shapes=[pltpu.VMEM((mb, MOL, H), jnp.float32)],
    )(node_t, gbias, wih_f, whh_f, bih_f, bhh_f,
      wih_b, whh_b, bih_b, bhh_b, wo, wob)


def kernel(f_atoms, f_bonds, a2b, b2a, b2revb, a_scope, params):
    p = params

    a2b_p = jnp.pad(a2b, ((0, NAp - NA1), (0, 0)))
    a2b2d = a2b_p.reshape(NW, APW * MAXNB // 128, 128)
    b2a2d = jnp.pad(b2a, (0, NBp - NB1)).reshape(NW, BNC, 128)
    b2revb2d = jnp.pad(b2revb, (0, NBp - NB1)).reshape(NW, BNC, 128)

    input_atom = _proj_relu(f_atoms, p['W_i_atom'], NAp, bm=1024)
    input_bond = _proj_relu(f_bonds, p['W_i_bond'], NBp, bm=1024)

    sc_agg_add = _make_sc_agg(True)
    sc_agg = _make_sc_agg(False)

    message_atom = input_atom
    message_bond = input_bond
    for d in range(2):
        message_atom = sc_agg_add(message_bond, a2b2d, message_atom)
        g1, g2 = _sc_bond_gather(message_atom, message_bond, b2a2d, b2revb2d)
        message_bond = _bond_update(g1, g2, input_bond, p['W_h_%d' % d])

    agg_f = sc_agg(message_bond, a2b2d, message_atom)
    node = _lr_matmul(agg_f, message_atom, input_atom, p['lr'])

    node3 = node[1:NA1].reshape(N_MOLS, MOL, H)
    mol_vecs = _gru_readout(
        node3,
        p['gru_bias'].reshape(1, H),
        p['gru_Wih_f'], p['gru_Whh_f'],
        p['gru_bih_f'].reshape(1, 3 * H), p['gru_bhh_f'].reshape(1, 3 * H),
        p['gru_Wih_b'], p['gru_Whh_b'],
        p['gru_bih_b'].reshape(1, 3 * H), p['gru_bhh_b'].reshape(1, 3 * H),
        p['W_o_w'], p['W_o_b'].reshape(1, H),
    )
    return mol_vecs.reshape(N_MOLS, H)


# speedup vs baseline: 1.2417x; 1.0016x over previous
import functools

import jax
import jax.numpy as jnp
from jax import lax
from jax.experimental import pallas as pl
from jax.experimental.pallas import tpu as pltpu
from jax.experimental.pallas import tpu_sc as plsc

H = 128
N_MOLS = 1000
MOL = 50
NA1 = N_MOLS * MOL + 1
NB1 = 150001
MAXNB = 6

NAp = 51200
NBp = 151552

NW = 32
APW = NAp // NW
AC = 64
ANC = APW // AC
AIR = (AC * MAXNB) // 128
BPW = NBp // NW
BC = 128
BNC = BPW // BC

_SC_MESH = plsc.VectorSubcoreMesh(core_axis_name="c", subcore_axis_name="s",
                                  num_cores=2, num_subcores=16)


def _wid():
    return lax.axis_index("s") * 2 + lax.axis_index("c")


def _sc_agg_body(add, bond_hbm, a2b_hbm, atom_hbm, out_hbm,
                 idx_v, gbuf0, gbuf1, abuf, obuf, sem0, sem1, sem2):
    w = _wid()
    pltpu.sync_copy(a2b_hbm.at[w], idx_v)
    gbufs = (gbuf0, gbuf1)
    sems = (sem0, sem1)

    def start(c, b):
        return [pltpu.async_copy(bond_hbm.at[idx_v.at[c * AIR + j]],
                                 gbufs[b].at[pl.ds(j * 128, 128)], sems[b])
                for j in range(AIR)]

    def finish(c, b):
        for j in range(AIR):
            pltpu.make_async_copy(bond_hbm.at[idx_v.at[c * AIR + j]],
                                  gbufs[b].at[pl.ds(j * 128, 128)],
                                  sems[b]).wait()

    def compute(c, b):
        abase = w * APW + c * AC
        if add:
            pltpu.async_copy(atom_hbm.at[pl.ds(abase, AC)], abuf, sem2).wait()
        gbuf = gbufs[b]

        @plsc.parallel_loop(0, AC, step=1, unroll=8)
        def atom_body(a):
            r0 = a * MAXNB
            for c8 in range(H // 16):
                sl = pl.ds(c8 * 16, 16)
                v0 = gbuf[r0 + 0, sl]
                v1 = gbuf[r0 + 1, sl]
                v2 = gbuf[r0 + 2, sl]
                v3 = gbuf[r0 + 3, sl]
                v4 = gbuf[r0 + 4, sl]
                v5 = gbuf[r0 + 5, sl]
                s = ((v0 + v4) + v2) + ((v1 + v5) + v3)
                m = jnp.maximum(jnp.maximum(jnp.maximum(v0, v1),
                                            jnp.maximum(v2, v3)),
                                jnp.maximum(v4, v5))
                o = s * m
                if add:
                    o = o + abuf[a, sl]
                obuf[a, sl] = o
        pltpu.sync_copy(obuf, out_hbm.at[pl.ds(abase, AC)])

    start(0, 0)

    def pair(i, carry):
        c0 = 2 * i
        c1 = c0 + 1

        @pl.when(c1 < ANC)
        def _():
            start(c1, 1)
        finish(c0, 0)
        compute(c0, 0)

        @pl.when(c0 + 2 < ANC)
        def _():
            start(c0 + 2, 0)

        @pl.when(c1 < ANC)
        def _():
            finish(c1, 1)
            compute(c1, 1)
        return carry

    lax.fori_loop(0, (ANC + 1) // 2, pair, 0)


def _make_sc_agg(add):
    body = functools.partial(_sc_agg_body, add)
    return pl.kernel(
        body,
        out_type=jax.ShapeDtypeStruct((NAp, H), jnp.float32),
        mesh=_SC_MESH,
        scratch_types=[
            pltpu.VMEM((ANC * AIR, 128), jnp.int32),
            pltpu.VMEM((AIR * 128, H), jnp.float32),
            pltpu.VMEM((AIR * 128, H), jnp.float32),
            pltpu.VMEM((AC, H), jnp.float32),
            pltpu.VMEM((AC, H), jnp.float32),
            pltpu.SemaphoreType.DMA,
            pltpu.SemaphoreType.DMA,
            pltpu.SemaphoreType.DMA,
        ],
    )


def _sc_bond_gather_body(atom_hbm, bond_hbm, b2a_hbm, b2revb_hbm,
                         o1_hbm, o2_hbm, ia, ir,
                         g10, g20, g11, g21, sem0, sem1, osem0, osem1):
    w = _wid()
    pltpu.sync_copy(b2a_hbm.at[w], ia)
    pltpu.sync_copy(b2revb_hbm.at[w], ir)
    g1s = (g10, g11)
    g2s = (g20, g21)
    sems = (sem0, sem1)
    osems = (osem0, osem1)

    def start(c, b):
        pltpu.async_copy(atom_hbm.at[ia.at[c]], g1s[b], sems[b])
        pltpu.async_copy(bond_hbm.at[ir.at[c]], g2s[b], sems[b])

    def finish_out(c, b):
        bbase = w * BPW + c * BC
        pltpu.make_async_copy(g1s[b], o1_hbm.at[pl.ds(bbase, BC)], osems[b]).wait()
        pltpu.make_async_copy(g2s[b], o2_hbm.at[pl.ds(bbase, BC)], osems[b]).wait()

    def emit(c, b):
        bbase = w * BPW + c * BC
        pltpu.make_async_copy(atom_hbm.at[ia.at[c]], g1s[b], sems[b]).wait()
        pltpu.make_async_copy(bond_hbm.at[ir.at[c]], g2s[b], sems[b]).wait()
        pltpu.async_copy(g1s[b], o1_hbm.at[pl.ds(bbase, BC)], osems[b])
        pltpu.async_copy(g2s[b], o2_hbm.at[pl.ds(bbase, BC)], osems[b])

    start(0, 0)
    start(1, 1)

    def pair(i, carry):
        c0 = 2 * i
        c1 = c0 + 1
        emit(c0, 0)

        @pl.when(c0 + 2 < BNC)
        def _():
            finish_out(c0, 0)
            start(c0 + 2, 0)

        @pl.when(c1 < BNC)
        def _():
            emit(c1, 1)

        @pl.when(c1 + 2 < BNC)
        def _():
            finish_out(c1, 1)
            start(c1 + 2, 1)
        return carry

    lax.fori_loop(0, (BNC + 1) // 2, pair, 0)
    finish_out(BNC - 2, (BNC - 2) % 2)
    finish_out(BNC - 1, (BNC - 1) % 2)


_sc_bond_gather = pl.kernel(
    _sc_bond_gather_body,
    out_type=(jax.ShapeDtypeStruct((NBp, H), jnp.float32),
              jax.ShapeDtypeStruct((NBp, H), jnp.float32)),
    mesh=_SC_MESH,
    scratch_types=[
        pltpu.VMEM((BNC, 128), jnp.int32),
        pltpu.VMEM((BNC, 128), jnp.int32),
        pltpu.VMEM((BC, H), jnp.float32),
        pltpu.VMEM((BC, H), jnp.float32),
        pltpu.VMEM((BC, H), jnp.float32),
        pltpu.VMEM((BC, H), jnp.float32),
        pltpu.SemaphoreType.DMA,
        pltpu.SemaphoreType.DMA,
        pltpu.SemaphoreType.DMA,
        pltpu.SemaphoreType.DMA,
    ],
)


def _dotT(x, w):
    return lax.dot_general(x, w, (((1,), (1,)), ((), ())),
                           preferred_element_type=jnp.float32)


def _proj_relu(x, w, np_, bm):
    n, k = x.shape
    nmain = n // bm
    tail = jnp.zeros((np_ - nmain * bm, k), x.dtype).at[:n - nmain * bm].set(
        x[nmain * bm:])

    def body(x_ref, t_ref, w_ref, o_ref):
        i = pl.program_id(0)

        @pl.when(i < nmain)
        def _():
            o_ref[...] = jax.nn.relu(_dotT(x_ref[...], w_ref[...]))

        @pl.when(i >= nmain)
        def _():
            o_ref[...] = jax.nn.relu(_dotT(t_ref[...], w_ref[...]))

    return pl.pallas_call(
        body,
        grid=(np_ // bm,),
        in_specs=[
            pl.BlockSpec((bm, k), lambda i: (jnp.minimum(i, nmain - 1), 0)),
            pl.BlockSpec((bm, k), lambda i: (jnp.maximum(i - nmain, 0), 0)),
            pl.BlockSpec((H, k), lambda i: (0, 0)),
        ],
        out_specs=pl.BlockSpec((bm, H), lambda i: (i, 0)),
        out_shape=jax.ShapeDtypeStruct((np_, H), jnp.float32),
    )(x, tail, w)


def _bond_update(g1, g2, ib, w, bm=1024):
    def body(g1_ref, g2_ref, ib_ref, w_ref, o_ref):
        mb = g1_ref[...] - g2_ref[...]
        o_ref[...] = jax.nn.relu(ib_ref[...] + _dotT(mb, w_ref[...]))

    return pl.pallas_call(
        body,
        grid=(NBp // bm,),
        in_specs=[pl.BlockSpec((bm, H), lambda i: (i, 0)),
                  pl.BlockSpec((bm, H), lambda i: (i, 0)),
                  pl.BlockSpec((bm, H), lambda i: (i, 0)),
                  pl.BlockSpec((H, H), lambda i: (0, 0))],
        out_specs=pl.BlockSpec((bm, H), lambda i: (i, 0)),
        out_shape=jax.ShapeDtypeStruct((NBp, H), jnp.float32),
    )(g1, g2, ib, w)


def _lr_matmul(agg, ma, ia, lr_w, bm=1024):
    def body(a_ref, m_ref, i_ref, w_ref, o_ref):
        cat = jnp.concatenate([a_ref[...], m_ref[...], i_ref[...]], axis=1)
        o_ref[...] = _dotT(cat, w_ref[...])

    return pl.pallas_call(
        body,
        grid=(NAp // bm,),
        in_specs=[pl.BlockSpec((bm, H), lambda i: (i, 0)),
                  pl.BlockSpec((bm, H), lambda i: (i, 0)),
                  pl.BlockSpec((bm, H), lambda i: (i, 0)),
                  pl.BlockSpec((H, 3 * H), lambda i: (0, 0))],
        out_specs=pl.BlockSpec((bm, H), lambda i: (i, 0)),
        out_shape=jax.ShapeDtypeStruct((NAp, H), jnp.float32),
    )(agg, ma, ia, lr_w)


def _gru_readout(node_t, gbias, wih_f, whh_f, bih_f, bhh_f,
                 wih_b, whh_b, bih_b, bhh_b, wo, wob, mb=500):
    def body(nd_ref, gb_ref, wif_ref, whf_ref, bif_ref, bhf_ref,
             wib_ref, whb_ref, bib_ref, bhb_ref, wo_ref, wob_ref,
             o_ref, fwd_ref):
        gb = gb_ref[...]

        def xt_at(t):
            x = jnp.reshape(nd_ref[:, pl.ds(t, 1), :], (mb, H))
            return jax.nn.relu(x + gb)

        def gru_step(xt, h, wi, wh, bi, bh):
            gi = _dotT(xt, wi[...]) + bi[...]
            gh = _dotT(h, wh[...]) + bh[...]
            r = jax.nn.sigmoid(gi[:, 0:H] + gh[:, 0:H])
            z = jax.nn.sigmoid(gi[:, H:2 * H] + gh[:, H:2 * H])
            n = jnp.tanh(gi[:, 2 * H:3 * H] + r * gh[:, 2 * H:3 * H])
            return (1.0 - z) * n + z * h

        h0 = jnp.max(nd_ref[...], axis=1)

        def fstep(t, h):
            h2 = gru_step(xt_at(t), h, wif_ref, whf_ref, bif_ref, bhf_ref)
            fwd_ref[:, pl.ds(t, 1), :] = jnp.reshape(h2, (mb, 1, H))
            return h2

        lax.fori_loop(0, MOL, fstep, h0)

        wo = wo_ref[...]
        wob = wob_ref[...]

        def bstep(i, carry):
            h, acc = carry
            t = MOL - 1 - i
            h2 = gru_step(xt_at(t), h, wib_ref, whb_ref, bib_ref, bhb_ref)
            fwd_t = jnp.reshape(fwd_ref[:, pl.ds(t, 1), :], (mb, H))
            cat = jnp.concatenate([fwd_t, h2], axis=1)
            ah = jax.nn.relu(_dotT(cat, wo) + wob)
            return h2, acc + ah

        _, acc = lax.fori_loop(0, MOL, bstep,
                               (h0, jnp.zeros((mb, H), jnp.float32)))
        o_ref[...] = jnp.reshape(acc * (1.0 / MOL), (1, mb, H))

    return pl.pallas_call(
        body,
        grid=(N_MOLS // mb,),
        in_specs=[pl.BlockSpec((mb, MOL, H), lambda i: (i, 0, 0)),
                  pl.BlockSpec((1, H), lambda i: (0, 0)),
                  pl.BlockSpec((3 * H, H), lambda i: (0, 0)),
                  pl.BlockSpec((3 * H, H), lambda i: (0, 0)),
                  pl.BlockSpec((1, 3 * H), lambda i: (0, 0)),
                  pl.BlockSpec((1, 3 * H), lambda i: (0, 0)),
                  pl.BlockSpec((3 * H, H), lambda i: (0, 0)),
                  pl.BlockSpec((3 * H, H), lambda i: (0, 0)),
                  pl.BlockSpec((1, 3 * H), lambda i: (0, 0)),
                  pl.BlockSpec((1, 3 * H), lambda i: (0, 0)),
                  pl.BlockSpec((H, 2 * H), lambda i: (0, 0)),
                  pl.BlockSpec((1, H), lambda i: (0, 0))],
        out_specs=pl.BlockSpec((1, mb, H), lambda i: (i, 0, 0)),
        out_shape=jax.ShapeDtypeStruct((N_MOLS // mb, mb, H), jnp.float32),
        scratch_shapes=[pltpu.VMEM((mb, MOL, H), jnp.float32)],
    )(node_t, gbias, wih_f, whh_f, bih_f, bhh_f,
      wih_b, whh_b, bih_b, bhh_b, wo, wob)


def kernel(f_atoms, f_bonds, a2b, b2a, b2revb, a_scope, params):
    p = params

    a2b_p = jnp.pad(a2b, ((0, NAp - NA1), (0, 0)))
    a2b2d = a2b_p.reshape(NW, APW * MAXNB // 128, 128)
    b2a2d = jnp.pad(b2a, (0, NBp - NB1)).reshape(NW, BNC, 128)
    b2revb2d = jnp.pad(b2revb, (0, NBp - NB1)).reshape(NW, BNC, 128)

    input_atom = _proj_relu(f_atoms, p['W_i_atom'], NAp, bm=1024)
    input_bond = _proj_relu(f_bonds, p['W_i_bond'], NBp, bm=1024)

    sc_agg_add = _make_sc_agg(True)
    sc_agg = _make_sc_agg(False)

    message_atom = input_atom
    message_bond = input_bond
    for d in range(2):
        message_atom = sc_agg_add(message_bond, a2b2d, message_atom)
        g1, g2 = _sc_bond_gather(message_atom, message_bond, b2a2d, b2revb2d)
        message_bond = _bond_update(g1, g2, input_bond, p['W_h_%d' % d])

    agg_f = sc_agg(message_bond, a2b2d, message_atom)
    node = _lr_matmul(agg_f, message_atom, input_atom, p['lr'])

    node3 = node[1:NA1].reshape(N_MOLS, MOL, H)
    mol_vecs = _gru_readout(
        node3,
        p['gru_bias'].reshape(1, H),
        p['gru_Wih_f'], p['gru_Whh_f'],
        p['gru_bih_f'].reshape(1, 3 * H), p['gru_bhh_f'].reshape(1, 3 * H),
        p['gru_Wih_b'], p['gru_Whh_b'],
        p['gru_bih_b'].reshape(1, 3 * H), p['gru_bhh_b'].reshape(1, 3 * H),
        p['W_o_w'], p['W_o_b'].reshape(1, H),
    )
    return mol_vecs.reshape(N_MOLS, H)


# speedup vs baseline: 1.2552x; 1.0109x over previous
import functools

import jax
import jax.numpy as jnp
from jax import lax
from jax.experimental import pallas as pl
from jax.experimental.pallas import tpu as pltpu
from jax.experimental.pallas import tpu_sc as plsc

H = 128
N_MOLS = 1000
MOL = 50
NA1 = N_MOLS * MOL + 1
NB1 = 150001
MAXNB = 6

NAp = 51200
NBp = 151552

NW = 32
APW = NAp // NW
AC = 64
ANC = APW // AC
AIR = (AC * MAXNB) // 128
BPW = NBp // NW
BC = 128
BNC = BPW // BC

_SC_MESH = plsc.VectorSubcoreMesh(core_axis_name="c", subcore_axis_name="s",
                                  num_cores=2, num_subcores=16)


def _wid():
    return lax.axis_index("s") * 2 + lax.axis_index("c")


def _sc_agg_body(add, bond_hbm, a2b_hbm, atom_hbm, out_hbm,
                 idx_v, gbuf0, gbuf1, abuf, obuf, sem0, sem1, sem2):
    w = _wid()
    pltpu.sync_copy(a2b_hbm.at[w], idx_v)
    gbufs = (gbuf0, gbuf1)
    sems = (sem0, sem1)

    def start(c, b):
        return [pltpu.async_copy(bond_hbm.at[idx_v.at[c * AIR + j]],
                                 gbufs[b].at[pl.ds(j * 128, 128)], sems[b])
                for j in range(AIR)]

    def finish(c, b):
        for j in range(AIR):
            pltpu.make_async_copy(bond_hbm.at[idx_v.at[c * AIR + j]],
                                  gbufs[b].at[pl.ds(j * 128, 128)],
                                  sems[b]).wait()

    def compute(c, b):
        abase = w * APW + c * AC
        if add:
            pltpu.async_copy(atom_hbm.at[pl.ds(abase, AC)], abuf, sem2).wait()
        gbuf = gbufs[b]

        @plsc.parallel_loop(0, AC, step=1, unroll=8)
        def atom_body(a):
            r0 = a * MAXNB
            for c8 in range(H // 16):
                sl = pl.ds(c8 * 16, 16)
                v0 = gbuf[r0 + 0, sl]
                v1 = gbuf[r0 + 1, sl]
                v2 = gbuf[r0 + 2, sl]
                v3 = gbuf[r0 + 3, sl]
                v4 = gbuf[r0 + 4, sl]
                v5 = gbuf[r0 + 5, sl]
                s = ((v0 + v4) + v2) + ((v1 + v5) + v3)
                m = jnp.maximum(jnp.maximum(jnp.maximum(v0, v1),
                                            jnp.maximum(v2, v3)),
                                jnp.maximum(v4, v5))
                o = s * m
                if add:
                    o = o + abuf[a, sl]
                obuf[a, sl] = o
        pltpu.sync_copy(obuf, out_hbm.at[pl.ds(abase, AC)])

    start(0, 0)

    def pair(i, carry):
        c0 = 2 * i
        c1 = c0 + 1

        @pl.when(c1 < ANC)
        def _():
            start(c1, 1)
        finish(c0, 0)
        compute(c0, 0)

        @pl.when(c0 + 2 < ANC)
        def _():
            start(c0 + 2, 0)

        @pl.when(c1 < ANC)
        def _():
            finish(c1, 1)
            compute(c1, 1)
        return carry

    lax.fori_loop(0, (ANC + 1) // 2, pair, 0)


def _make_sc_agg(add):
    body = functools.partial(_sc_agg_body, add)
    return pl.kernel(
        body,
        out_type=jax.ShapeDtypeStruct((NAp, H), jnp.float32),
        mesh=_SC_MESH,
        scratch_types=[
            pltpu.VMEM((ANC * AIR, 128), jnp.int32),
            pltpu.VMEM((AIR * 128, H), jnp.float32),
            pltpu.VMEM((AIR * 128, H), jnp.float32),
            pltpu.VMEM((AC, H), jnp.float32),
            pltpu.VMEM((AC, H), jnp.float32),
            pltpu.SemaphoreType.DMA,
            pltpu.SemaphoreType.DMA,
            pltpu.SemaphoreType.DMA,
        ],
    )


def _sc_bond_gather_body(atom_hbm, bond_hbm, b2a_hbm, b2revb_hbm,
                         o1_hbm, o2_hbm, ia, ir,
                         g10, g20, g11, g21, sem0, sem1, osem0, osem1):
    w = _wid()
    pltpu.sync_copy(b2a_hbm.at[w], ia)
    pltpu.sync_copy(b2revb_hbm.at[w], ir)
    g1s = (g10, g11)
    g2s = (g20, g21)
    sems = (sem0, sem1)
    osems = (osem0, osem1)

    def start(c, b):
        pltpu.async_copy(atom_hbm.at[ia.at[c]], g1s[b], sems[b])
        pltpu.async_copy(bond_hbm.at[ir.at[c]], g2s[b], sems[b])

    def finish_out(c, b):
        bbase = w * BPW + c * BC
        pltpu.make_async_copy(g1s[b], o1_hbm.at[pl.ds(bbase, BC)], osems[b]).wait()
        pltpu.make_async_copy(g2s[b], o2_hbm.at[pl.ds(bbase, BC)], osems[b]).wait()

    def emit(c, b):
        bbase = w * BPW + c * BC
        pltpu.make_async_copy(atom_hbm.at[ia.at[c]], g1s[b], sems[b]).wait()
        pltpu.make_async_copy(bond_hbm.at[ir.at[c]], g2s[b], sems[b]).wait()
        pltpu.async_copy(g1s[b], o1_hbm.at[pl.ds(bbase, BC)], osems[b])
        pltpu.async_copy(g2s[b], o2_hbm.at[pl.ds(bbase, BC)], osems[b])

    start(0, 0)
    start(1, 1)

    def pair(i, carry):
        c0 = 2 * i
        c1 = c0 + 1
        emit(c0, 0)

        @pl.when(c0 + 2 < BNC)
        def _():
            finish_out(c0, 0)
            start(c0 + 2, 0)

        @pl.when(c1 < BNC)
        def _():
            emit(c1, 1)

        @pl.when(c1 + 2 < BNC)
        def _():
            finish_out(c1, 1)
            start(c1 + 2, 1)
        return carry

    lax.fori_loop(0, (BNC + 1) // 2, pair, 0)
    finish_out(BNC - 2, (BNC - 2) % 2)
    finish_out(BNC - 1, (BNC - 1) % 2)


_sc_bond_gather = pl.kernel(
    _sc_bond_gather_body,
    out_type=(jax.ShapeDtypeStruct((NBp, H), jnp.float32),
              jax.ShapeDtypeStruct((NBp, H), jnp.float32)),
    mesh=_SC_MESH,
    scratch_types=[
        pltpu.VMEM((BNC, 128), jnp.int32),
        pltpu.VMEM((BNC, 128), jnp.int32),
        pltpu.VMEM((BC, H), jnp.float32),
        pltpu.VMEM((BC, H), jnp.float32),
        pltpu.VMEM((BC, H), jnp.float32),
        pltpu.VMEM((BC, H), jnp.float32),
        pltpu.SemaphoreType.DMA,
        pltpu.SemaphoreType.DMA,
        pltpu.SemaphoreType.DMA,
        pltpu.SemaphoreType.DMA,
    ],
)


def _dotT(x, w):
    return lax.dot_general(x, w, (((1,), (1,)), ((), ())),
                           preferred_element_type=jnp.float32)


def _proj_relu(x, w, np_, bm):
    n, k = x.shape
    nmain = n // bm
    tail = jnp.zeros((np_ - nmain * bm, k), x.dtype).at[:n - nmain * bm].set(
        x[nmain * bm:])

    def body(x_ref, t_ref, w_ref, o_ref):
        i = pl.program_id(0)

        @pl.when(i < nmain)
        def _():
            o_ref[...] = jax.nn.relu(_dotT(x_ref[...], w_ref[...]))

        @pl.when(i >= nmain)
        def _():
            o_ref[...] = jax.nn.relu(_dotT(t_ref[...], w_ref[...]))

    return pl.pallas_call(
        body,
        grid=(np_ // bm,),
        in_specs=[
            pl.BlockSpec((bm, k), lambda i: (jnp.minimum(i, nmain - 1), 0)),
            pl.BlockSpec((bm, k), lambda i: (jnp.maximum(i - nmain, 0), 0)),
            pl.BlockSpec((H, k), lambda i: (0, 0)),
        ],
        out_specs=pl.BlockSpec((bm, H), lambda i: (i, 0)),
        out_shape=jax.ShapeDtypeStruct((np_, H), jnp.float32),
    )(x, tail, w)


def _bond_update(g1, g2, ib, w, bm=1024):
    def body(g1_ref, g2_ref, ib_ref, w_ref, o_ref):
        mb = g1_ref[...] - g2_ref[...]
        o_ref[...] = jax.nn.relu(ib_ref[...] + _dotT(mb, w_ref[...]))

    return pl.pallas_call(
        body,
        grid=(NBp // bm,),
        in_specs=[pl.BlockSpec((bm, H), lambda i: (i, 0)),
                  pl.BlockSpec((bm, H), lambda i: (i, 0)),
                  pl.BlockSpec((bm, H), lambda i: (i, 0)),
                  pl.BlockSpec((H, H), lambda i: (0, 0))],
        out_specs=pl.BlockSpec((bm, H), lambda i: (i, 0)),
        out_shape=jax.ShapeDtypeStruct((NBp, H), jnp.float32),
    )(g1, g2, ib, w)


def _lr_matmul(agg, ma, ia, lr_w, bm=1024):
    def body(a_ref, m_ref, i_ref, w_ref, o_ref):
        cat = jnp.concatenate([a_ref[...], m_ref[...], i_ref[...]], axis=1)
        o_ref[...] = _dotT(cat, w_ref[...])

    return pl.pallas_call(
        body,
        grid=(NAp // bm,),
        in_specs=[pl.BlockSpec((bm, H), lambda i: (i, 0)),
                  pl.BlockSpec((bm, H), lambda i: (i, 0)),
                  pl.BlockSpec((bm, H), lambda i: (i, 0)),
                  pl.BlockSpec((H, 3 * H), lambda i: (0, 0))],
        out_specs=pl.BlockSpec((bm, H), lambda i: (i, 0)),
        out_shape=jax.ShapeDtypeStruct((NAp, H), jnp.float32),
    )(agg, ma, ia, lr_w)


def _gru_readout(node_t, gbias, wih_f, whh_f, bih_f, bhh_f,
                 wih_b, whh_b, bih_b, bhh_b, wo, wob, mb=500):
    def body(nd_ref, gb_ref, wif_ref, whf_ref, bif_ref, bhf_ref,
             wib_ref, whb_ref, bib_ref, bhb_ref, wo_ref, wob_ref,
             o_ref, fwd_ref, tm_ref):
        gb = gb_ref[...]

        def stage(t, carry):
            tm_ref[pl.ds(t, 1)] = jnp.reshape(
                nd_ref[:, pl.ds(t, 1), :], (1, mb, H))
            return carry
        lax.fori_loop(0, MOL, stage, 0)

        def xt_at(t):
            x = jnp.reshape(tm_ref[pl.ds(t, 1)], (mb, H))
            return jax.nn.relu(x + gb)

        def gru_step(xt, h, wi, wh, bi, bh):
            gi = _dotT(xt, wi[...]) + bi[...]
            gh = _dotT(h, wh[...]) + bh[...]
            r = jax.nn.sigmoid(gi[:, 0:H] + gh[:, 0:H])
            z = jax.nn.sigmoid(gi[:, H:2 * H] + gh[:, H:2 * H])
            n = jnp.tanh(gi[:, 2 * H:3 * H] + r * gh[:, 2 * H:3 * H])
            return (1.0 - z) * n + z * h

        h0 = jnp.max(nd_ref[...], axis=1)

        def fstep(t, h):
            h2 = gru_step(xt_at(t), h, wif_ref, whf_ref, bif_ref, bhf_ref)
            fwd_ref[pl.ds(t, 1)] = jnp.reshape(h2, (1, mb, H))
            return h2

        lax.fori_loop(0, MOL, fstep, h0)

        wo = wo_ref[...]
        wob = wob_ref[...]

        def bstep(i, carry):
            h, acc = carry
            t = MOL - 1 - i
            h2 = gru_step(xt_at(t), h, wib_ref, whb_ref, bib_ref, bhb_ref)
            fwd_t = jnp.reshape(fwd_ref[pl.ds(t, 1)], (mb, H))
            cat = jnp.concatenate([fwd_t, h2], axis=1)
            ah = jax.nn.relu(_dotT(cat, wo) + wob)
            return h2, acc + ah

        _, acc = lax.fori_loop(0, MOL, bstep,
                               (h0, jnp.zeros((mb, H), jnp.float32)))
        o_ref[...] = jnp.reshape(acc * (1.0 / MOL), (1, mb, H))

    return pl.pallas_call(
        body,
        grid=(N_MOLS // mb,),
        in_specs=[pl.BlockSpec((mb, MOL, H), lambda i: (i, 0, 0)),
                  pl.BlockSpec((1, H), lambda i: (0, 0)),
                  pl.BlockSpec((3 * H, H), lambda i: (0, 0)),
                  pl.BlockSpec((3 * H, H), lambda i: (0, 0)),
                  pl.BlockSpec((1, 3 * H), lambda i: (0, 0)),
                  pl.BlockSpec((1, 3 * H), lambda i: (0, 0)),
                  pl.BlockSpec((3 * H, H), lambda i: (0, 0)),
                  pl.BlockSpec((3 * H, H), lambda i: (0, 0)),
                  pl.BlockSpec((1, 3 * H), lambda i: (0, 0)),
                  pl.BlockSpec((1, 3 * H), lambda i: (0, 0)),
                  pl.BlockSpec((H, 2 * H), lambda i: (0, 0)),
                  pl.BlockSpec((1, H), lambda i: (0, 0))],
        out_specs=pl.BlockSpec((1, mb, H), lambda i: (i, 0, 0)),
        out_shape=jax.ShapeDtypeStruct((N_MOLS // mb, mb, H), jnp.float32),
        scratch_shapes=[pltpu.VMEM((MOL, mb, H), jnp.float32),
                        pltpu.VMEM((MOL, mb, H), jnp.float32)],
    )(node_t, gbias, wih_f, whh_f, bih_f, bhh_f,
      wih_b, whh_b, bih_b, bhh_b, wo, wob)


def kernel(f_atoms, f_bonds, a2b, b2a, b2revb, a_scope, params):
    p = params

    a2b_p = jnp.pad(a2b, ((0, NAp - NA1), (0, 0)))
    a2b2d = a2b_p.reshape(NW, APW * MAXNB // 128, 128)
    b2a2d = jnp.pad(b2a, (0, NBp - NB1)).reshape(NW, BNC, 128)
    b2revb2d = jnp.pad(b2revb, (0, NBp - NB1)).reshape(NW, BNC, 128)

    input_atom = _proj_relu(f_atoms, p['W_i_atom'], NAp, bm=1024)
    input_bond = _proj_relu(f_bonds, p['W_i_bond'], NBp, bm=1024)

    sc_agg_add = _make_sc_agg(True)
    sc_agg = _make_sc_agg(False)

    message_atom = input_atom
    message_bond = input_bond
    for d in range(2):
        message_atom = sc_agg_add(message_bond, a2b2d, message_atom)
        g1, g2 = _sc_bond_gather(message_atom, message_bond, b2a2d, b2revb2d)
        message_bond = _bond_update(g1, g2, input_bond, p['W_h_%d' % d])

    agg_f = sc_agg(message_bond, a2b2d, message_atom)
    node = _lr_matmul(agg_f, message_atom, input_atom, p['lr'])

    node3 = node[1:NA1].reshape(N_MOLS, MOL, H)
    mol_vecs = _gru_readout(
        node3,
        p['gru_bias'].reshape(1, H),
        p['gru_Wih_f'], p['gru_Whh_f'],
        p['gru_bih_f'].reshape(1, 3 * H), p['gru_bhh_f'].reshape(1, 3 * H),
        p['gru_Wih_b'], p['gru_Whh_b'],
        p['gru_bih_b'].reshape(1, 3 * H), p['gru_bhh_b'].reshape(1, 3 * H),
        p['W_o_w'], p['W_o_b'].reshape(1, H),
    )
    return mol_vecs.reshape(N_MOLS, H)
